# baseline (speedup 1.0000x reference)
import functools

import jax
import jax.numpy as jnp
from jax.experimental import pallas as pl
from jax.experimental.pallas import tpu as pltpu
from jax.experimental.pallas import tpu_sc as plsc

N_NODES = 10000
N_EDGES = 320000
LATENT = 64
GDIM = 8

NC, NS = 2, 16
NW = NC * NS
CHUNK = 128
SLOTS = 4
E_PAD = 327680
NP = 10240
BN = 2048
NBN = NP // BN
N_ACC = NP
ROWS_PER_TILE = N_ACC // NS
TSTAGE_PER_TILE = 2 * NP // NS


def _relu(x):
    return jnp.maximum(x, 0.0)


def _ln(x, scale, bias, eps=1e-6):
    m = jnp.mean(x, axis=-1, keepdims=True)
    xc = x - m
    v = jnp.mean(xc * xc, axis=-1, keepdims=True)
    return xc * jax.lax.rsqrt(v + eps) * scale + bias


def _enc_body(nodes_ref, wenc_ref, benc_ref, wproj_ref, gmat_ref, wg_ref,
              brows_ref, n_ref, p_ref, cvec_ref):
    n = jnp.dot(nodes_ref[...], wenc_ref[...],
                preferred_element_type=jnp.float32) + benc_ref[...]
    n_ref[...] = n
    p_ref[...] = jnp.dot(n, wproj_ref[...],
                         preferred_element_type=jnp.float32)
    c = jnp.dot(gmat_ref[...], wg_ref[...],
                preferred_element_type=jnp.float32) + brows_ref[...]
    cvec_ref[...] = jnp.concatenate([c, c], axis=1)


def _encoder_call(nodes, wenc, benc, wproj, gmat, wg, brows):
    full = lambda i: (0, 0)
    return pl.pallas_call(
        _enc_body,
        grid=(NBN,),
        in_specs=[
            pl.BlockSpec((BN, 128), lambda i: (i, 0)),
            pl.BlockSpec((128, LATENT), full),
            pl.BlockSpec((1, LATENT), full),
            pl.BlockSpec((LATENT, 2 * LATENT), full),
            pl.BlockSpec((4, 4 * GDIM), full),
            pl.BlockSpec((4 * GDIM, LATENT), full),
            pl.BlockSpec((4, LATENT), full),
        ],
        out_specs=[
            pl.BlockSpec((BN, LATENT), lambda i: (i, 0)),
            pl.BlockSpec((BN, 2 * LATENT), lambda i: (i, 0)),
            pl.BlockSpec((4, 2 * LATENT), full),
        ],
        out_shape=[
            jax.ShapeDtypeStruct((NP, LATENT), jnp.float32),
            jax.ShapeDtypeStruct((NP, 2 * LATENT), jnp.float32),
            jax.ShapeDtypeStruct((4, 2 * LATENT), jnp.float32),
        ],
    )(nodes, wenc, benc, wproj, gmat, wg, brows)


BE2 = 2048
NBE2 = (E_PAD // 2) // BE2
EPAIR = N_EDGES // 2


def _edge_body(has_e, ce_row, gs_ref, gr_ref, e_ref, w1e_ref, w2_ref,
               b2_ref, w3_ref, b3_ref, ce_ref, out_ref):
    x = gs_ref[...] + gr_ref[...] + ce_ref[ce_row:ce_row + 1, :]
    if has_e:
        x = x + jnp.dot(e_ref[...], w1e_ref[...],
                        preferred_element_type=jnp.float32)
    h1 = _relu(x)
    h2 = _relu(jnp.dot(h1, w2_ref[...],
                       preferred_element_type=jnp.float32) + b2_ref[...])
    y = jnp.dot(h2, w3_ref[...],
                preferred_element_type=jnp.float32) + b3_ref[...]
    rows = pl.program_id(0) * BE2 + jax.lax.broadcasted_iota(
        jnp.int32, (BE2, 1), 0)
    out_ref[...] = jnp.where(rows < EPAIR, y, 0.0)


def _edge_call(gs2, gr2, e_prev, w1e, w2, b2, w3, b3, cvec, ce_row):
    has_e = e_prev is not None
    full = lambda i: (0, 0)
    blk = lambda i: (i, 0)
    D = 2 * LATENT
    in_specs = [
        pl.BlockSpec((BE2, D), blk),
        pl.BlockSpec((BE2, D), blk),
    ]
    args = [gs2, gr2]
    if has_e:
        in_specs += [pl.BlockSpec((BE2, D), blk),
                     pl.BlockSpec((D, D), full)]
        args += [e_prev, w1e]
    in_specs += [
        pl.BlockSpec((D, D), full),
        pl.BlockSpec((1, D), full),
        pl.BlockSpec((D, D), full),
        pl.BlockSpec((1, D), full),
        pl.BlockSpec((4, D), full),
    ]
    args += [w2, b2, w3, b3, cvec]
    body = functools.partial(_edge_body, has_e, ce_row)
    if not has_e:
        def body(gs, gr, w2r, b2r, w3r, b3r, cer, outr):
            _edge_body(False, ce_row, gs, gr, None, None, w2r, b2r, w3r,
                       b3r, cer, outr)
    return pl.pallas_call(
        body,
        grid=(NBE2,),
        in_specs=in_specs,
        out_specs=pl.BlockSpec((BE2, D), blk),
        out_shape=jax.ShapeDtypeStruct((E_PAD // 2, D), jnp.float32),
    )(*args)


def _node_body(final, cn_row, n_ref, r0_ref, r1_ref, a1_ref, bmat_ref,
               w2_ref, b2_ref, w3_ref, b3_ref, cn_ref, lns_ref, lnb_ref,
               wnext_ref, bnext_ref, *out_refs):
    n = n_ref[...]
    recv = r0_ref[...] + r1_ref[...]
    m1 = _relu(jnp.dot(n, a1_ref[...], preferred_element_type=jnp.float32)
               + jnp.dot(recv, bmat_ref[...],
                         preferred_element_type=jnp.float32)
               + cn_ref[cn_row:cn_row + 1, :LATENT])
    m2 = _relu(jnp.dot(m1, w2_ref[...],
                       preferred_element_type=jnp.float32) + b2_ref[...])
    nn = jnp.dot(m2, w3_ref[...],
                 preferred_element_type=jnp.float32) + b3_ref[...]
    y = _ln(nn + n, lns_ref[...], lnb_ref[...])
    if final:
        out_refs[0][...] = jnp.dot(
            y, wnext_ref[...], preferred_element_type=jnp.float32
        ) + bnext_ref[...]
    else:
        out_refs[0][...] = y
        out_refs[1][...] = jnp.dot(y, wnext_ref[...],
                                   preferred_element_type=jnp.float32)


def _node_call(final, n, r0, r1, a1, bmat, w2, b2, w3, b3, cvec, cn_row,
               lns, lnb, wnext, bnext):
    full = lambda i: (0, 0)
    blk = lambda i: (i, 0)
    next_cols = 128 if final else 2 * LATENT
    in_specs = [
        pl.BlockSpec((BN, LATENT), blk),
        pl.BlockSpec((BN, LATENT), blk),
        pl.BlockSpec((BN, LATENT), blk),
        pl.BlockSpec((LATENT, LATENT), full),
        pl.BlockSpec((LATENT, LATENT), full),
        pl.BlockSpec((LATENT, LATENT), full),
        pl.BlockSpec((1, LATENT), full),
        pl.BlockSpec((LATENT, LATENT), full),
        pl.BlockSpec((1, LATENT), full),
        pl.BlockSpec((4, 2 * LATENT), full),
        pl.BlockSpec((1, LATENT), full),
        pl.BlockSpec((1, LATENT), full),
        pl.BlockSpec((LATENT, next_cols), full),
        pl.BlockSpec((1, 128), full),
    ]
    if final:
        out_specs = pl.BlockSpec((BN, 128), blk)
        out_shape = jax.ShapeDtypeStruct((NP, 128), jnp.float32)
    else:
        out_specs = [pl.BlockSpec((BN, LATENT), blk),
                     pl.BlockSpec((BN, 2 * LATENT), blk)]
        out_shape = [
            jax.ShapeDtypeStruct((NP, LATENT), jnp.float32),
            jax.ShapeDtypeStruct((NP, 2 * LATENT), jnp.float32)]
    return pl.pallas_call(
        functools.partial(_node_body, final, cn_row),
        grid=(NBN,),
        in_specs=in_specs,
        out_specs=out_specs,
        out_shape=out_shape,
    )(n, r0, r1, a1, bmat, w2, b2, w3, b3, cvec, lns, lnb, wnext, bnext)


def _sc_mesh():
    return plsc.VectorSubcoreMesh(core_axis_name="c", subcore_axis_name="s",
                                  num_cores=NC, num_subcores=NS)


def _gather_rows(table, idx2):
    n_chunks = (2 * E_PAD) // (NW * CHUNK)
    n_iters = n_chunks // SLOTS

    def body(t_hbm, idx_hbm, out_hbm, tsh, *scr):
        ibufs = scr[:SLOTS]
        rbufs = scr[SLOTS:2 * SLOTS]
        sems = scr[2 * SLOTS:3 * SLOTS]
        sid = jax.lax.axis_index("s")
        wid = sid * NC + jax.lax.axis_index("c")
        t0 = sid * TSTAGE_PER_TILE
        pltpu.sync_copy(t_hbm.at[pl.ds(t0, TSTAGE_PER_TILE)],
                        tsh.at[pl.ds(t0, TSTAGE_PER_TILE)])
        plsc.subcore_barrier()
        base = wid * n_chunks * CHUNK

        def it(i, carry):
            offs = [base + (i * SLOTS + k) * CHUNK for k in range(SLOTS)]
            his = [pltpu.async_copy(idx_hbm.at[pl.ds(offs[k], CHUNK)],
                                    ibufs[k], sems[k]) for k in range(SLOTS)]
            hgs = []
            for k in range(SLOTS):
                his[k].wait()
                hgs.append(pltpu.async_copy(tsh.at[ibufs[k]], rbufs[k],
                                            sems[k]))
            hss = []
            for k in range(SLOTS):
                hgs[k].wait()
                hss.append(pltpu.async_copy(
                    rbufs[k], out_hbm.at[pl.ds(offs[k], CHUNK)], sems[k]))
            for k in range(SLOTS):
                hss[k].wait()
            return carry

        jax.lax.fori_loop(0, n_iters, it, 0)

    scratch = ([pltpu.VMEM((CHUNK,), jnp.int32) for _ in range(SLOTS)]
               + [pltpu.VMEM((CHUNK, LATENT), jnp.float32)
                  for _ in range(SLOTS)]
               + [pltpu.SemaphoreType.DMA for _ in range(SLOTS)])
    return pl.kernel(
        body,
        out_type=jax.ShapeDtypeStruct((2 * E_PAD, LATENT), jnp.float32),
        mesh=_sc_mesh(),
        scratch_types=[pltpu.VMEM_SHARED((2 * NP, LATENT), jnp.float32)]
        + scratch,
        compiler_params=pltpu.CompilerParams(use_tc_tiling_on_sc=False),
    )(table, idx2)


def _segment_partials(e_new, receivers_p, zeros):
    n_chunks = E_PAD // (NW * CHUNK)
    n_iters = n_chunks // SLOTS

    def body(e_hbm, idx_hbm, z_hbm, out_hbm, acc, *scr):
        ibufs = scr[:SLOTS]
        rbufs = scr[SLOTS:2 * SLOTS]
        isems = scr[2 * SLOTS:3 * SLOTS]
        rsems = scr[3 * SLOTS:4 * SLOTS]
        cid = jax.lax.axis_index("c")
        sid = jax.lax.axis_index("s")
        wid = sid * NC + cid
        r0 = sid * ROWS_PER_TILE
        pltpu.sync_copy(z_hbm.at[pl.ds(r0, ROWS_PER_TILE)],
                        acc.at[pl.ds(r0, ROWS_PER_TILE)])
        plsc.subcore_barrier()
        base = wid * n_chunks * CHUNK

        def it(i, carry):
            offs = [base + (i * SLOTS + k) * CHUNK for k in range(SLOTS)]
            his = [pltpu.async_copy(idx_hbm.at[pl.ds(offs[k], CHUNK)],
                                    ibufs[k], isems[k]) for k in range(SLOTS)]
            hrs = [pltpu.async_copy(e_hbm.at[pl.ds(offs[k], CHUNK)],
                                    rbufs[k], rsems[k]) for k in range(SLOTS)]
            for k in range(SLOTS):
                his[k].wait()
                hrs[k].wait()
                pltpu.sync_copy(rbufs[k], acc.at[ibufs[k]], add=True)
            return carry

        jax.lax.fori_loop(0, n_iters, it, 0)
        plsc.subcore_barrier()
        pltpu.sync_copy(acc.at[pl.ds(r0, ROWS_PER_TILE)],
                        out_hbm.at[cid, pl.ds(r0, ROWS_PER_TILE)])

    scratch = ([pltpu.VMEM_SHARED((N_ACC, LATENT), jnp.float32)]
               + [pltpu.VMEM((CHUNK,), jnp.int32) for _ in range(SLOTS)]
               + [pltpu.VMEM((CHUNK, LATENT), jnp.float32)
                  for _ in range(SLOTS)]
               + [pltpu.SemaphoreType.DMA for _ in range(2 * SLOTS)])
    part = pl.kernel(
        body,
        out_type=jax.ShapeDtypeStruct((NC, N_ACC, LATENT), jnp.float32),
        mesh=_sc_mesh(),
        scratch_types=scratch,
        compiler_params=pltpu.CompilerParams(use_tc_tiling_on_sc=False),
    )(e_new, receivers_p, zeros)
    return part[0], part[1]


def kernel(nodes, senders, receivers, globals_, params):
    f32 = jnp.float32
    enc = params["encoder"]
    dec = params["decoder"]
    steps = params["steps"]

    pad = E_PAD - N_EDGES
    senders_p = jnp.concatenate(
        [senders, jnp.zeros((pad,), senders.dtype)]).astype(jnp.int32)
    receivers_p = jnp.concatenate(
        [receivers, jnp.zeros((pad,), receivers.dtype)]).astype(jnp.int32)
    idx2 = jnp.concatenate([2 * senders_p, 2 * receivers_p + 1])
    acczeros = jnp.zeros((N_ACC, LATENT), f32)

    g0 = globals_[0].astype(f32)

    e0W1 = steps[0]["edge_mlp"][0]["W"]
    w1s0, w1r0, w1g0 = e0W1[:64], e0W1[64:128], e0W1[128:136]
    e1W1 = steps[1]["edge_mlp"][0]["W"]
    w1e1, w1s1, w1r1, w1g1 = (e1W1[:64], e1W1[64:128], e1W1[128:192],
                              e1W1[192:200])
    n0W1 = steps[0]["node_mlp"][0]["W"]
    a10, bm0, wgn0 = n0W1[:64], n0W1[64:128], n0W1[128:136]
    n1W1 = steps[1]["node_mlp"][0]["W"]
    a11, bm1, wgn1 = n1W1[:64], n1W1[64:128], n1W1[128:136]

    gmat = jnp.zeros((4, 4 * GDIM), f32)
    gmat = gmat.at[0, 0:8].set(g0)
    gmat = gmat.at[1, 8:16].set(g0)
    gmat = gmat.at[2, 16:24].set(2.0 * g0)
    gmat = gmat.at[3, 24:32].set(2.0 * g0)
    wg_all = jnp.concatenate([w1g0, wgn0, w1g1, wgn1], axis=0)
    brows = jnp.stack([
        steps[0]["edge_mlp"][0]["b"], steps[0]["node_mlp"][0]["b"],
        steps[1]["edge_mlp"][0]["b"], steps[1]["node_mlp"][0]["b"]])

    row = lambda b: b.reshape(1, -1)

    def bdiag(w):
        z = jnp.zeros((2 * LATENT, 2 * LATENT), f32)
        return z.at[:LATENT, :LATENT].set(w).at[LATENT:, LATENT:].set(w)

    def brow(b):
        return jnp.tile(b.reshape(1, -1), (1, 2))

    wproj0 = jnp.concatenate([w1s0, w1r0], axis=1)
    wproj1 = jnp.concatenate([w1s1, w1r1], axis=1)

    nodes_p = jnp.concatenate(
        [nodes, jnp.zeros((NP - N_NODES, nodes.shape[1]), f32)])
    n0, p0, cvec = _encoder_call(
        nodes_p, enc["W"], row(enc["b"]), wproj0, gmat, wg_all, brows)

    g0rows = _gather_rows(p0.reshape(2 * NP, LATENT), idx2)
    gs0 = g0rows[:E_PAD].reshape(E_PAD // 2, 2 * LATENT)
    gr0 = g0rows[E_PAD:].reshape(E_PAD // 2, 2 * LATENT)
    em0 = steps[0]["edge_mlp"]
    e_new0 = _edge_call(gs0, gr0, None, None, bdiag(em0[1]["W"]),
                        brow(em0[1]["b"]), bdiag(em0[2]["W"]),
                        brow(em0[2]["b"]), cvec, 0)
    p0a, p0b = _segment_partials(
        e_new0.reshape(E_PAD, LATENT), receivers_p, acczeros)
    nm0 = steps[0]["node_mlp"]
    n1, p1 = _node_call(
        False, n0, p0a, p0b, a10, bm0, nm0[1]["W"], row(nm0[1]["b"]),
        nm0[2]["W"], row(nm0[2]["b"]), cvec, 1,
        row(steps[0]["ln_scale"]), row(steps[0]["ln_bias"]),
        wproj1, jnp.zeros((1, 128), f32))

    g1rows = _gather_rows(p1.reshape(2 * NP, LATENT), idx2)
    gs1 = g1rows[:E_PAD].reshape(E_PAD // 2, 2 * LATENT)
    gr1 = g1rows[E_PAD:].reshape(E_PAD // 2, 2 * LATENT)
    em1 = steps[1]["edge_mlp"]
    e_new1 = _edge_call(gs1, gr1, e_new0, bdiag(w1e1), bdiag(em1[1]["W"]),
                        brow(em1[1]["b"]), bdiag(em1[2]["W"]),
                        brow(em1[2]["b"]), cvec, 2)
    p1a, p1b = _segment_partials(
        e_new1.reshape(E_PAD, LATENT), receivers_p, acczeros)
    nm1 = steps[1]["node_mlp"]
    out = _node_call(
        True, n1, p1a, p1b, a11, bm1, nm1[1]["W"], row(nm1[1]["b"]),
        nm1[2]["W"], row(nm1[2]["b"]), cvec, 3,
        row(steps[1]["ln_scale"]), row(steps[1]["ln_bias"]),
        dec["W"], row(dec["b"]))
    return out[:N_NODES]
